# baseline (speedup 1.0000x reference)
import jax
import jax.numpy as jnp
from jax import lax
from jax.experimental import pallas as pl
from jax.experimental.pallas import tpu as pltpu
from jax.experimental.pallas import tpu_sc as plsc

N_NODES = 100000
N_EDGES = 6400000
LANES = 16
NC = 2
NS = 16
NW = NC * NS
ROW_W = 128
CHUNK_ROWS = 16
CHUNK = CHUNK_ROWS * ROW_W
PER_W = 98
N_CHUNKS_PAD = NW * PER_W
E_PAD = N_CHUNKS_PAD * CHUNK
GROUPS = CHUNK // LANES
ACC_TILE_ROWS = 6256
ACC_LAST_ROWS = N_NODES - 15 * ACC_TILE_ROWS

_MAGIC = 0x5F3759DF

_DNUMS = lax.GatherDimensionNumbers(
    offset_dims=(), collapsed_slice_dims=(0,), start_index_map=(0,))


def _vgather(table_vec, idx):
  return lax.gather(table_vec, idx[:, None], _DNUMS, (1,),
                    mode=lax.GatherScatterMode.PROMISE_IN_BOUNDS)


def _force_kernel(xs_h, ys_h, zs_h, ct_h, praw, zrows, dst_f, src_f,
                  out,
                  di1a, si1a, di1b, si1b,
                  gxia, gyia, gzia, gcta, gxja, gyja, gzja,
                  gxib, gyib, gzib, gctb, gxjb, gyjb, gzjb,
                  di2a, di2b,
                  stxa, stya, stza, stxb, styb, stzb,
                  praw_v, xs, ys, zs, cts, accx, accy, accz,
                  semi_a, semi_b, semg_a, semg_b, sems_a, sems_b):
  cid_c = lax.axis_index("c")
  sid = lax.axis_index("s")
  wid = sid * NC + cid_c

  r0_off = sid * ACC_TILE_ROWS

  def _stage(total):
    done = 0
    while done < total:
      n = min(2048, total - done)
      sl = pl.ds(r0_off + done, n)
      zsl = pl.ds(0, n)
      pltpu.sync_copy(xs_h.at[sl], xs.at[sl])
      pltpu.sync_copy(ys_h.at[sl], ys.at[sl])
      pltpu.sync_copy(zs_h.at[sl], zs.at[sl])
      pltpu.sync_copy(ct_h.at[sl], cts.at[sl])
      pltpu.sync_copy(zrows.at[zsl], accx.at[sl])
      pltpu.sync_copy(zrows.at[zsl], accy.at[sl])
      pltpu.sync_copy(zrows.at[zsl], accz.at[sl])
      done += n

  @pl.when(sid < 15)
  def _():
    _stage(ACC_TILE_ROWS)

  @pl.when(sid == 15)
  def _():
    _stage(ACC_LAST_ROWS)

  pltpu.sync_copy(praw, praw_v)
  k_rep = praw_v[0]
  r0_t = praw_v[1]
  kadh = praw_v[2]
  r_on = praw_v[3]
  delta = praw_v[4]
  mu_f = praw_v[5]
  A_v = mu_f * k_rep
  B_v = mu_f * kadh
  invd_v = 1.0 / jnp.maximum(delta, 1e-8)

  plsc.subcore_barrier()

  def fire_front(g, di1, si1, semi):
    ebase = (wid + NW * g) * CHUNK
    pltpu.async_copy(dst_f.at[pl.ds(ebase, CHUNK)], di1, semi)
    pltpu.async_copy(src_f.at[pl.ds(ebase, CHUNK)], si1, semi)

  def wait_front(di1, si1, semi):
    pltpu.make_async_copy(dst_f.at[pl.ds(0, CHUNK)], di1, semi).wait()
    pltpu.make_async_copy(src_f.at[pl.ds(0, CHUNK)], si1, semi).wait()

  def fire_gathers(di1, si1, bufs, semg):
    gxi, gyi, gzi, gct, gxj, gyj, gzj = bufs
    pltpu.async_copy(xs.at[di1], gxi, semg)
    pltpu.async_copy(ys.at[di1], gyi, semg)
    pltpu.async_copy(zs.at[di1], gzi, semg)
    pltpu.async_copy(cts.at[di1], gct, semg)
    pltpu.async_copy(xs.at[si1], gxj, semg)
    pltpu.async_copy(ys.at[si1], gyj, semg)
    pltpu.async_copy(zs.at[si1], gzj, semg)

  def wait_gathers(di1, si1, bufs, semg):
    gxi, gyi, gzi, gct, gxj, gyj, gzj = bufs
    pltpu.make_async_copy(xs.at[di1], gxi, semg).wait()
    pltpu.make_async_copy(ys.at[di1], gyi, semg).wait()
    pltpu.make_async_copy(zs.at[di1], gzi, semg).wait()
    pltpu.make_async_copy(cts.at[di1], gct, semg).wait()
    pltpu.make_async_copy(xs.at[si1], gxj, semg).wait()
    pltpu.make_async_copy(ys.at[si1], gyj, semg).wait()
    pltpu.make_async_copy(zs.at[si1], gzj, semg).wait()

  def compute(di1, si1, di2, bufs, stx, sty, stz):
    gxi, gyi, gzi, gct, gxj, gyj, gzj = bufs

    def _group(j, _):
      l = j * LANES
      sl = pl.ds(l, LANES)
      m = j // (ROW_W // LANES)
      lofs = (j % (ROW_W // LANES)) * LANES
      xi = gxi[sl]
      yi = gyi[sl]
      zi = gzi[sl]
      ct = gct[sl]
      xj = gxj[sl]
      yj = gyj[sl]
      zj = gzj[sl]
      dv = di1[sl]
      sv = si1[sl]
      di2[m, pl.ds(lofs, LANES)] = dv

      dx = xj - xi
      dy = yj - yi
      dz = zj - zi
      r2 = dx * dx + dy * dy + dz * dz
      r2s = jnp.maximum(r2, 1e-30)
      yv = plsc.bitcast(_MAGIC - (plsc.bitcast(r2s, jnp.int32) >> 1),
                        jnp.float32)
      h = 0.5 * r2s
      yv = yv * (1.5 - h * yv * yv)
      yv = yv * (1.5 - h * yv * yv)
      r = r2s * yv
      inv_rs = jnp.minimum(yv, 1e8)

      A = _vgather(A_v, ct)
      B = _vgather(B_v, ct)
      r0v = _vgather(r0_t, ct)
      ronv = _vgather(r_on, ct)
      invd = _vgather(invd_v, ct)

      rel = r - r0v
      frep = A * jnp.maximum(-rel, 0.0)
      e1 = jnp.exp(-(rel * invd))
      e2 = jnp.exp((r - ronv) * invd)
      den = (1.0 + e1) * (1.0 + e2)
      coef = (B * rel / den - frep) * inv_rs
      coef = jnp.where(sv == dv, 0.0, coef)

      stx[sl] = coef * dx
      sty[sl] = coef * dy
      stz[sl] = coef * dz
      return 0

    lax.fori_loop(0, GROUPS, _group, 0)

  def fire_scat(di2, stx, sty, stz, sems):
    def _scat(m, _):
      idx = di2.at[m]
      rsl = pl.ds(m * ROW_W, ROW_W)
      pltpu.async_copy(stx.at[rsl], accx.at[idx], sems, add=True)
      pltpu.async_copy(sty.at[rsl], accy.at[idx], sems, add=True)
      pltpu.async_copy(stz.at[rsl], accz.at[idx], sems, add=True)
      return 0
    lax.fori_loop(0, CHUNK_ROWS, _scat, 0)

  def drain_scat(di2, stx, sty, stz, sems):
    def _sdrain(m, _):
      idx = di2.at[m]
      rsl = pl.ds(m * ROW_W, ROW_W)
      pltpu.make_async_copy(stx.at[rsl], accx.at[idx], sems).wait()
      pltpu.make_async_copy(sty.at[rsl], accy.at[idx], sems).wait()
      pltpu.make_async_copy(stz.at[rsl], accz.at[idx], sems).wait()
      return 0
    lax.fori_loop(0, CHUNK_ROWS, _sdrain, 0)

  bufs_a = (gxia, gyia, gzia, gcta, gxja, gyja, gzja)
  bufs_b = (gxib, gyib, gzib, gctb, gxjb, gyjb, gzjb)
  A_set = (di1a, si1a, bufs_a, di2a, stxa, stya, stza, semi_a, semg_a, sems_a)
  B_set = (di1b, si1b, bufs_b, di2b, stxb, styb, stzb, semi_b, semg_b, sems_b)

  def phase(g_next, cur, nxt, fire_next, drain_first):
    (di1, si1, bufs, di2, stx, sty, stz, semi, semg, sems) = cur
    (ndi1, nsi1, nbufs, _, _, _, _, nsemi, nsemg, _) = nxt
    if fire_next:
      fire_front(g_next, ndi1, nsi1, nsemi)
    if drain_first:
      drain_scat(di2, stx, sty, stz, sems)
    wait_gathers(di1, si1, bufs, semg)
    if fire_next:
      wait_front(ndi1, nsi1, nsemi)
      fire_gathers(ndi1, nsi1, nbufs, nsemg)
    compute(di1, si1, di2, bufs, stx, sty, stz)
    fire_scat(di2, stx, sty, stz, sems)

  fire_front(0, di1a, si1a, semi_a)
  wait_front(di1a, si1a, semi_a)
  fire_gathers(di1a, si1a, bufs_a, semg_a)

  phase(1, A_set, B_set, True, False)
  phase(2, B_set, A_set, True, False)

  def _pair(h, _):
    g = 2 * h
    phase(g + 1, A_set, B_set, True, True)
    phase(g + 2, B_set, A_set, True, True)
    return 0
  lax.fori_loop(1, PER_W // 2 - 1, _pair, 0)

  phase(PER_W - 1, A_set, B_set, True, True)
  phase(0, B_set, A_set, False, True)

  drain_scat(di2a, stxa, stya, stza, sems_a)
  drain_scat(di2b, stxb, styb, stzb, sems_b)

  plsc.subcore_barrier()

  def _dump(total):
    done = 0
    while done < total:
      n = min(2048, total - done)
      sl = pl.ds(r0_off + done, n)
      pltpu.sync_copy(accx.at[sl], out.at[pl.ds(cid_c * 3 * N_NODES + r0_off + done, n)])
      pltpu.sync_copy(accy.at[sl], out.at[pl.ds((cid_c * 3 + 1) * N_NODES + r0_off + done, n)])
      pltpu.sync_copy(accz.at[sl], out.at[pl.ds((cid_c * 3 + 2) * N_NODES + r0_off + done, n)])
      done += n

  @pl.when(sid < 15)
  def _():
    _dump(ACC_TILE_ROWS)

  @pl.when(sid == 15)
  def _():
    _dump(ACC_LAST_ROWS)


@jax.jit
def _run(xs, ys, zs, cts, praw, zrows, dst_f, src_f):
  mesh = plsc.VectorSubcoreMesh(core_axis_name="c", subcore_axis_name="s",
                                num_cores=NC, num_subcores=NS)
  vi = lambda: pltpu.VMEM((CHUNK,), jnp.int32)
  vf = lambda: pltpu.VMEM((CHUNK,), jnp.float32)
  f = pl.kernel(
      _force_kernel,
      out_type=jax.ShapeDtypeStruct((2 * 3 * N_NODES,), jnp.float32),
      mesh=mesh,
      compiler_params=pltpu.CompilerParams(needs_layout_passes=False,
                                           use_tc_tiling_on_sc=False),
      scratch_types=[
          vi(), vi(), vi(), vi(),
          vf(), vf(), vf(), vi(), vf(), vf(), vf(),
          vf(), vf(), vf(), vi(), vf(), vf(), vf(),
          pltpu.VMEM((CHUNK_ROWS, ROW_W), jnp.int32),
          pltpu.VMEM((CHUNK_ROWS, ROW_W), jnp.int32),
          vf(), vf(), vf(), vf(), vf(), vf(),
          pltpu.VMEM((8, LANES), jnp.float32),
          pltpu.VMEM_SHARED((N_NODES,), jnp.float32),
          pltpu.VMEM_SHARED((N_NODES,), jnp.float32),
          pltpu.VMEM_SHARED((N_NODES,), jnp.float32),
          pltpu.VMEM_SHARED((N_NODES,), jnp.int32),
          pltpu.VMEM_SHARED((N_NODES,), jnp.float32),
          pltpu.VMEM_SHARED((N_NODES,), jnp.float32),
          pltpu.VMEM_SHARED((N_NODES,), jnp.float32),
          pltpu.SemaphoreType.DMA,
          pltpu.SemaphoreType.DMA,
          pltpu.SemaphoreType.DMA,
          pltpu.SemaphoreType.DMA,
          pltpu.SemaphoreType.DMA,
          pltpu.SemaphoreType.DMA,
      ],
  )
  return f(xs, ys, zs, cts, praw, zrows, dst_f, src_f)


def kernel(pos, p, cell_type, edge_index):
  xs = pos[:, 0]
  ys = pos[:, 1]
  zs = pos[:, 2]
  cts = cell_type.astype(jnp.int32)
  praw = jnp.zeros((8, LANES), jnp.float32).at[:6, :5].set(p.T)
  zrows = jnp.zeros((2048,), jnp.float32)
  pad = jnp.zeros((E_PAD - N_EDGES,), edge_index.dtype)
  dst_f = jnp.concatenate([edge_index[0], pad])
  src_f = jnp.concatenate([edge_index[1], pad])
  out = _run(xs, ys, zs, cts, praw, zrows, dst_f, src_f)
  o = out.reshape(2, 3, N_NODES)
  return (o[0] + o[1]).T


# speedup vs baseline: 1.1507x; 1.1507x over previous
import jax
import jax.numpy as jnp
from jax import lax
from jax.experimental import pallas as pl
from jax.experimental.pallas import tpu as pltpu
from jax.experimental.pallas import tpu_sc as plsc

N_NODES = 100000
N_EDGES = 6400000
LANES = 16
NC = 2
NS = 16
NW = NC * NS
ROW_W = 128
CHUNK_ROWS = 16
CHUNK = CHUNK_ROWS * ROW_W
PER_W = 98
N_CHUNKS_PAD = NW * PER_W
E_PAD = N_CHUNKS_PAD * CHUNK
GROUPS = CHUNK // LANES
ACC_TILE_ROWS = 6256
ACC_LAST_ROWS = N_NODES - 15 * ACC_TILE_ROWS

_MAGIC = 0x5F3759DF

_DNUMS = lax.GatherDimensionNumbers(
    offset_dims=(), collapsed_slice_dims=(0,), start_index_map=(0,))


def _vgather(table_vec, idx):
  return lax.gather(table_vec, idx[:, None], _DNUMS, (1,),
                    mode=lax.GatherScatterMode.PROMISE_IN_BOUNDS)


def _force_kernel(xs_h, ys_h, zct_h, praw, zrows, dst_f, src_f,
                  out,
                  di1a, si1a, di1b, si1b,
                  gxia, gyia, gzia, gxja, gyja, gzja,
                  gxib, gyib, gzib, gxjb, gyjb, gzjb,
                  di2a, di2b,
                  stxa, stya, stza, stxb, styb, stzb,
                  praw_v, xs, ys, zct, accx, accy, accz,
                  semi_a, semi_b, semg_a, semg_b, sems_a, sems_b):
  cid_c = lax.axis_index("c")
  sid = lax.axis_index("s")
  wid = sid * NC + cid_c

  r0_off = sid * ACC_TILE_ROWS

  def _stage(total):
    done = 0
    while done < total:
      n = min(2048, total - done)
      sl = pl.ds(r0_off + done, n)
      zsl = pl.ds(0, n)
      pltpu.sync_copy(xs_h.at[sl], xs.at[sl])
      pltpu.sync_copy(ys_h.at[sl], ys.at[sl])
      pltpu.sync_copy(zct_h.at[sl], zct.at[sl])
      pltpu.sync_copy(zrows.at[zsl], accx.at[sl])
      pltpu.sync_copy(zrows.at[zsl], accy.at[sl])
      pltpu.sync_copy(zrows.at[zsl], accz.at[sl])
      done += n

  @pl.when(sid < 15)
  def _():
    _stage(ACC_TILE_ROWS)

  @pl.when(sid == 15)
  def _():
    _stage(ACC_LAST_ROWS)

  pltpu.sync_copy(praw, praw_v)
  k_rep = praw_v[0]
  r0_t = praw_v[1]
  kadh = praw_v[2]
  r_on = praw_v[3]
  delta = praw_v[4]
  mu_f = praw_v[5]
  A_v = mu_f * k_rep
  B_v = mu_f * kadh
  invd_v = 1.0 / jnp.maximum(delta, 1e-8)

  plsc.subcore_barrier()

  def fire_front(g, di1, si1, semi):
    ebase = (wid + NW * g) * CHUNK
    pltpu.async_copy(dst_f.at[pl.ds(ebase, CHUNK)], di1, semi)
    pltpu.async_copy(src_f.at[pl.ds(ebase, CHUNK)], si1, semi)

  def wait_front(di1, si1, semi):
    pltpu.make_async_copy(dst_f.at[pl.ds(0, CHUNK)], di1, semi).wait()
    pltpu.make_async_copy(src_f.at[pl.ds(0, CHUNK)], si1, semi).wait()

  def fire_gathers(di1, si1, bufs, semg):
    gxi, gyi, gzi, gxj, gyj, gzj = bufs
    pltpu.async_copy(xs.at[di1], gxi, semg)
    pltpu.async_copy(ys.at[di1], gyi, semg)
    pltpu.async_copy(zct.at[di1], gzi, semg)
    pltpu.async_copy(xs.at[si1], gxj, semg)
    pltpu.async_copy(ys.at[si1], gyj, semg)
    pltpu.async_copy(zct.at[si1], gzj, semg)

  def wait_gathers(di1, si1, bufs, semg):
    gxi, gyi, gzi, gxj, gyj, gzj = bufs
    pltpu.make_async_copy(xs.at[di1], gxi, semg).wait()
    pltpu.make_async_copy(ys.at[di1], gyi, semg).wait()
    pltpu.make_async_copy(zct.at[di1], gzi, semg).wait()
    pltpu.make_async_copy(xs.at[si1], gxj, semg).wait()
    pltpu.make_async_copy(ys.at[si1], gyj, semg).wait()
    pltpu.make_async_copy(zct.at[si1], gzj, semg).wait()

  def compute(di1, si1, di2, bufs, stx, sty, stz):
    gxi, gyi, gzi, gxj, gyj, gzj = bufs

    def _group(j, _):
      l = j * LANES
      sl = pl.ds(l, LANES)
      m = j // (ROW_W // LANES)
      lofs = (j % (ROW_W // LANES)) * LANES
      xi = gxi[sl]
      yi = gyi[sl]
      zib = plsc.bitcast(gzi[sl], jnp.int32)
      ct = zib & 7
      zi = plsc.bitcast(zib & -8, jnp.float32)
      xj = gxj[sl]
      yj = gyj[sl]
      zj = plsc.bitcast(plsc.bitcast(gzj[sl], jnp.int32) & -8, jnp.float32)
      dv = di1[sl]
      sv = si1[sl]
      di2[m, pl.ds(lofs, LANES)] = dv

      dx = xj - xi
      dy = yj - yi
      dz = zj - zi
      r2 = dx * dx + dy * dy + dz * dz
      r2s = jnp.maximum(r2, 1e-30)
      yv = plsc.bitcast(_MAGIC - (plsc.bitcast(r2s, jnp.int32) >> 1),
                        jnp.float32)
      h = 0.5 * r2s
      yv = yv * (1.5 - h * yv * yv)
      yv = yv * (1.5 - h * yv * yv)
      yv = yv * (1.5 - h * yv * yv)
      r = r2s * yv
      inv_rs = jnp.minimum(yv, 1e8)

      A = _vgather(A_v, ct)
      B = _vgather(B_v, ct)
      r0v = _vgather(r0_t, ct)
      ronv = _vgather(r_on, ct)
      invd = _vgather(invd_v, ct)

      rel = r - r0v
      frep = A * jnp.maximum(-rel, 0.0)
      e1 = jnp.exp(-(rel * invd))
      e2 = jnp.exp((r - ronv) * invd)
      den = (1.0 + e1) * (1.0 + e2)
      coef = (B * rel / den - frep) * inv_rs
      coef = jnp.where(sv == dv, 0.0, coef)

      stx[sl] = coef * dx
      sty[sl] = coef * dy
      stz[sl] = coef * dz
      return 0

    lax.fori_loop(0, GROUPS, _group, 0)

  def fire_scat(di2, stx, sty, stz, sems):
    def _scat(m, _):
      idx = di2.at[m]
      rsl = pl.ds(m * ROW_W, ROW_W)
      pltpu.async_copy(stx.at[rsl], accx.at[idx], sems, add=True)
      pltpu.async_copy(sty.at[rsl], accy.at[idx], sems, add=True)
      pltpu.async_copy(stz.at[rsl], accz.at[idx], sems, add=True)
      return 0
    lax.fori_loop(0, CHUNK_ROWS, _scat, 0)

  def drain_scat(di2, stx, sty, stz, sems):
    def _sdrain(m, _):
      idx = di2.at[m]
      rsl = pl.ds(m * ROW_W, ROW_W)
      pltpu.make_async_copy(stx.at[rsl], accx.at[idx], sems).wait()
      pltpu.make_async_copy(sty.at[rsl], accy.at[idx], sems).wait()
      pltpu.make_async_copy(stz.at[rsl], accz.at[idx], sems).wait()
      return 0
    lax.fori_loop(0, CHUNK_ROWS, _sdrain, 0)

  bufs_a = (gxia, gyia, gzia, gxja, gyja, gzja)
  bufs_b = (gxib, gyib, gzib, gxjb, gyjb, gzjb)
  A_set = (di1a, si1a, bufs_a, di2a, stxa, stya, stza, semi_a, semg_a, sems_a)
  B_set = (di1b, si1b, bufs_b, di2b, stxb, styb, stzb, semi_b, semg_b, sems_b)

  def phase(g_next, cur, nxt, fire_next):
    (di1, si1, bufs, di2, stx, sty, stz, semi, semg, sems) = cur
    (ndi1, nsi1, nbufs, _, _, _, _, nsemi, nsemg, _) = nxt
    if fire_next:
      fire_front(g_next, ndi1, nsi1, nsemi)
    wait_gathers(di1, si1, bufs, semg)
    if fire_next:
      wait_front(ndi1, nsi1, nsemi)
      fire_gathers(ndi1, nsi1, nbufs, nsemg)
    compute(di1, si1, di2, bufs, stx, sty, stz)
    fire_scat(di2, stx, sty, stz, sems)
    drain_scat(di2, stx, sty, stz, sems)

  fire_front(0, di1a, si1a, semi_a)
  wait_front(di1a, si1a, semi_a)
  fire_gathers(di1a, si1a, bufs_a, semg_a)

  def _pair(h, _):
    g = 2 * h
    phase(g + 1, A_set, B_set, True)
    phase(g + 2, B_set, A_set, True)
    return 0
  lax.fori_loop(0, PER_W // 2 - 1, _pair, 0)

  phase(PER_W - 1, A_set, B_set, True)
  phase(0, B_set, A_set, False)

  plsc.subcore_barrier()

  def _dump(total):
    done = 0
    while done < total:
      n = min(2048, total - done)
      sl = pl.ds(r0_off + done, n)
      pltpu.sync_copy(accx.at[sl], out.at[pl.ds(cid_c * 3 * N_NODES + r0_off + done, n)])
      pltpu.sync_copy(accy.at[sl], out.at[pl.ds((cid_c * 3 + 1) * N_NODES + r0_off + done, n)])
      pltpu.sync_copy(accz.at[sl], out.at[pl.ds((cid_c * 3 + 2) * N_NODES + r0_off + done, n)])
      done += n

  @pl.when(sid < 15)
  def _():
    _dump(ACC_TILE_ROWS)

  @pl.when(sid == 15)
  def _():
    _dump(ACC_LAST_ROWS)


@jax.jit
def _run(xs, ys, zct, praw, zrows, dst_f, src_f):
  mesh = plsc.VectorSubcoreMesh(core_axis_name="c", subcore_axis_name="s",
                                num_cores=NC, num_subcores=NS)
  vi = lambda: pltpu.VMEM((CHUNK,), jnp.int32)
  vf = lambda: pltpu.VMEM((CHUNK,), jnp.float32)
  f = pl.kernel(
      _force_kernel,
      out_type=jax.ShapeDtypeStruct((2 * 3 * N_NODES,), jnp.float32),
      mesh=mesh,
      compiler_params=pltpu.CompilerParams(needs_layout_passes=False,
                                           use_tc_tiling_on_sc=False),
      scratch_types=[
          vi(), vi(), vi(), vi(),
          vf(), vf(), vf(), vf(), vf(), vf(),
          vf(), vf(), vf(), vf(), vf(), vf(),
          pltpu.VMEM((CHUNK_ROWS, ROW_W), jnp.int32),
          pltpu.VMEM((CHUNK_ROWS, ROW_W), jnp.int32),
          vf(), vf(), vf(), vf(), vf(), vf(),
          pltpu.VMEM((8, LANES), jnp.float32),
          pltpu.VMEM_SHARED((N_NODES,), jnp.float32),
          pltpu.VMEM_SHARED((N_NODES,), jnp.float32),
          pltpu.VMEM_SHARED((N_NODES,), jnp.float32),
          pltpu.VMEM_SHARED((N_NODES,), jnp.float32),
          pltpu.VMEM_SHARED((N_NODES,), jnp.float32),
          pltpu.VMEM_SHARED((N_NODES,), jnp.float32),
          pltpu.SemaphoreType.DMA,
          pltpu.SemaphoreType.DMA,
          pltpu.SemaphoreType.DMA,
          pltpu.SemaphoreType.DMA,
          pltpu.SemaphoreType.DMA,
          pltpu.SemaphoreType.DMA,
      ],
  )
  return f(xs, ys, zct, praw, zrows, dst_f, src_f)


def kernel(pos, p, cell_type, edge_index):
  xs = pos[:, 0]
  ys = pos[:, 1]
  zbits = lax.bitcast_convert_type(pos[:, 2], jnp.int32)
  zct = lax.bitcast_convert_type((zbits & -8) | cell_type.astype(jnp.int32),
                                 jnp.float32)
  praw = jnp.zeros((8, LANES), jnp.float32).at[:6, :5].set(p.T)
  zrows = jnp.zeros((2048,), jnp.float32)
  pad = jnp.zeros((E_PAD - N_EDGES,), edge_index.dtype)
  dst_f = jnp.concatenate([edge_index[0], pad])
  src_f = jnp.concatenate([edge_index[1], pad])
  out = _run(xs, ys, zct, praw, zrows, dst_f, src_f)
  o = out.reshape(2, 3, N_NODES)
  return (o[0] + o[1]).T
